# baseline (speedup 1.0000x reference)
import functools

import jax
import jax.numpy as jnp
from jax import lax
from jax.experimental import pallas as pl
from jax.experimental.pallas import tpu as pltpu
from jax.experimental.pallas import tpu_sc as plsc

N = 10000
E = 320000
D = 128

NC = 2
NS = 16
NW = NC * NS
NP = 10240
RPT = NP // NS
K = 80
NCH = 128
EPT = NCH * K
EP = NW * EPT
DF = 4
HCH = NCH // 2

_mesh = plsc.VectorSubcoreMesh(core_axis_name="core", subcore_axis_name="subcore")


def _fill(ref, rows, value):
    cols = ref.shape[-1]
    vec = jnp.full((16,), value, dtype=ref.dtype)

    @pl.loop(0, rows)
    def _(i):
        @pl.loop(0, cols, step=16)
        def _(j):
            ref[i, pl.ds(j, 16)] = vec


def _fill_rows(ref, rows, vec16):
    zvec = jnp.zeros((16,), dtype=ref.dtype)

    @pl.loop(0, rows)
    def _(i):
        for j in range(8):
            ref[i, pl.ds(j * 16, 16)] = vec16 if j == 0 else zvec


@functools.partial(
    pl.kernel,
    out_type=jax.ShapeDtypeStruct((NC, NP, D), jnp.float32),
    mesh=_mesh,
    scratch_types=[
        pltpu.VMEM_SHARED((NP, D), jnp.float32),
        pltpu.VMEM((K, D), jnp.float32),
        pltpu.VMEM((K, D), jnp.float32),
        pltpu.VMEM((HCH, K), jnp.int32),
        pltpu.VMEM((HCH, K), jnp.int32),
        pltpu.SemaphoreType.DMA,
    ],
)
def _deg_kernel(src3_hbm, dst3_hbm, cnt_hbm, cnt_sh, ones_s, ones_d,
                sidx, didx, dsem):
    c = lax.axis_index("core")
    s = lax.axis_index("subcore")
    wid = c * NS + s
    _fill(ones_s, K, 0.0)

    @pl.loop(0, RPT // K)
    def _(z):
        pltpu.sync_copy(ones_s, cnt_sh.at[pl.ds(s * RPT + z * K, K)])

    lane = lax.broadcasted_iota(jnp.int32, (16,), 0)
    _fill_rows(ones_s, K, jnp.where(lane == 0, 1.0, 0.0).astype(jnp.float32))
    _fill_rows(ones_d, K, jnp.where(lane == 1, 1.0, 0.0).astype(jnp.float32))
    plsc.subcore_barrier()

    for half in range(2):
        base = half * HCH
        pltpu.sync_copy(src3_hbm.at[wid, pl.ds(base, HCH)], sidx)
        pltpu.sync_copy(dst3_hbm.at[wid, pl.ds(base, HCH)], didx)

        @pl.loop(0, HCH // DF)
        def _(u):
            descs = []
            for q in range(DF):
                ci = u * DF + q
                descs.append(pltpu.async_copy(
                    ones_s, cnt_sh.at[sidx.at[ci]], dsem, add=True))
                descs.append(pltpu.async_copy(
                    ones_d, cnt_sh.at[didx.at[ci]], dsem, add=True))
            for dd in descs:
                dd.wait()

    plsc.subcore_barrier()
    rows = pl.ds(s * RPT, RPT)
    pltpu.sync_copy(cnt_sh.at[rows], cnt_hbm.at[c, rows])


@functools.partial(
    pl.kernel,
    out_type=jax.ShapeDtypeStruct((NC, NP, D), jnp.float32),
    mesh=_mesh,
    scratch_types=[
        pltpu.VMEM_SHARED((NP, D), jnp.float32),
        pltpu.VMEM((2, K, D), jnp.float32),
        pltpu.VMEM((EPT,), jnp.int32),
        pltpu.VMEM((NCH, K), jnp.int32),
        pltpu.SemaphoreType.DMA,
        pltpu.SemaphoreType.DMA,
        pltpu.SemaphoreType.DMA,
    ],
)
def _agg_kernel(h_hbm, src1_hbm, dst3_hbm, pagg_hbm, agg_sh, rows_v,
                sidx, didx, isem, gsem, ssem):
    c = lax.axis_index("core")
    s = lax.axis_index("subcore")
    wid = c * NS + s
    lds = pltpu.async_copy(src1_hbm.at[pl.ds(wid * EPT, EPT)], sidx, isem)
    ldd = pltpu.async_copy(dst3_hbm.at[wid], didx, isem)

    _fill(rows_v.at[0], K, 0.0)

    @pl.loop(0, RPT // K)
    def _(z):
        pltpu.sync_copy(rows_v.at[0], agg_sh.at[pl.ds(s * RPT + z * K, K)])

    lds.wait()
    ldd.wait()
    plsc.subcore_barrier()

    def g_start(ci, b):
        return pltpu.async_copy(h_hbm.at[sidx.at[pl.ds(ci * K, K)]],
                                rows_v.at[b], gsem)

    def s_start(ci, b):
        return pltpu.async_copy(rows_v.at[b], agg_sh.at[didx.at[ci]], ssem,
                                add=True)

    g_start(0, 0).wait()

    @pl.loop(0, NCH - 1)
    def _(c):
        b = c % 2
        gd = g_start(c + 1, 1 - b)
        sd = s_start(c, b)
        sd.wait()
        gd.wait()

    s_start(NCH - 1, (NCH - 1) % 2).wait()

    plsc.subcore_barrier()
    rows = pl.ds(s * RPT, RPT)
    pltpu.sync_copy(agg_sh.at[rows], pagg_hbm.at[c, rows])


def _norm_body(deg_ref, ns_ref, nd_ref):
    cnt = deg_ref[0] + deg_ref[1]
    ns_ref[...] = lax.rsqrt(jnp.maximum(cnt[:, 0:1], 1.0))
    nd_ref[...] = lax.rsqrt(jnp.maximum(cnt[:, 1:2], 1.0))


def _norms(deg):
    return pl.pallas_call(
        _norm_body,
        out_shape=(jax.ShapeDtypeStruct((NP, 1), jnp.float32),
                   jax.ShapeDtypeStruct((NP, 1), jnp.float32)),
    )(deg)


_BR = 1024


def _mm1_body(x_ref, w_ref, ns_ref, o_ref):
    o_ref[...] = jnp.dot(x_ref[...], w_ref[...],
                         preferred_element_type=jnp.float32) * ns_ref[...]


def _mm1(x, W1, ns):
    return pl.pallas_call(
        _mm1_body,
        grid=(NP // _BR,),
        in_specs=[
            pl.BlockSpec((_BR, D), lambda i: (i, 0)),
            pl.BlockSpec((D, D), lambda i: (0, 0)),
            pl.BlockSpec((_BR, 1), lambda i: (i, 0)),
        ],
        out_specs=pl.BlockSpec((_BR, D), lambda i: (i, 0)),
        out_shape=jax.ShapeDtypeStruct((NP, D), jnp.float32),
    )(x, W1, ns)


def _mid_body(p_ref, nd_ref, b_ref, w_ref, ns_ref, o_ref):
    h = (p_ref[0] + p_ref[1]) * nd_ref[...] + b_ref[...]
    h = jnp.maximum(h, 0.0)
    o_ref[...] = jnp.dot(h, w_ref[...],
                         preferred_element_type=jnp.float32) * ns_ref[...]


def _mid(pagg, nd, b1, W2, ns):
    return pl.pallas_call(
        _mid_body,
        grid=(NP // _BR,),
        in_specs=[
            pl.BlockSpec((NC, _BR, D), lambda i: (0, i, 0)),
            pl.BlockSpec((_BR, 1), lambda i: (i, 0)),
            pl.BlockSpec((1, D), lambda i: (0, 0)),
            pl.BlockSpec((D, D), lambda i: (0, 0)),
            pl.BlockSpec((_BR, 1), lambda i: (i, 0)),
        ],
        out_specs=pl.BlockSpec((_BR, D), lambda i: (i, 0)),
        out_shape=jax.ShapeDtypeStruct((NP, D), jnp.float32),
    )(pagg, nd, b1, W2, ns)


def _fin_body(p_ref, nd_ref, b_ref, wt_ref, bfc_ref, o_ref):
    h = (p_ref[0] + p_ref[1]) * nd_ref[...] + b_ref[...]
    o_ref[...] = jnp.dot(h, wt_ref[...],
                         preferred_element_type=jnp.float32) + bfc_ref[...]


def _fin(pagg, nd, b2, WfcT, bfc):
    return pl.pallas_call(
        _fin_body,
        grid=(NP // _BR,),
        in_specs=[
            pl.BlockSpec((NC, _BR, D), lambda i: (0, i, 0)),
            pl.BlockSpec((_BR, 1), lambda i: (i, 0)),
            pl.BlockSpec((1, D), lambda i: (0, 0)),
            pl.BlockSpec((D, D), lambda i: (0, 0)),
            pl.BlockSpec((1, D), lambda i: (0, 0)),
        ],
        out_specs=pl.BlockSpec((_BR, D), lambda i: (i, 0)),
        out_shape=jax.ShapeDtypeStruct((N, D), jnp.float32),
    )(pagg, nd, b2, WfcT, bfc)


def kernel(in_feat, edge_index, W1, b1, W2, b2, Wfc, bfc):
    pad = jnp.full((2, EP - E), NP - 1, dtype=edge_index.dtype)
    ep = jnp.concatenate([edge_index, pad], axis=1)
    src1 = ep[0]
    src3 = ep[0].reshape(NW, NCH, K)
    dst3 = ep[1].reshape(NW, NCH, K)
    deg = _deg_kernel(src3, dst3)
    ns, nd = _norms(deg)
    h1 = _mm1(in_feat, W1, ns)
    p1 = _agg_kernel(h1, src1, dst3)
    h2 = _mid(p1, nd, b1.reshape(1, D), W2, ns)
    p2 = _agg_kernel(h2, src1, dst3)
    out = _fin(p2, nd, b2.reshape(1, D), Wfc.T, bfc.reshape(1, D))
    return out


# speedup vs baseline: 2.0958x; 2.0958x over previous
import functools

import jax
import jax.numpy as jnp
from jax import lax
from jax.experimental import pallas as pl
from jax.experimental.pallas import tpu as pltpu
from jax.experimental.pallas import tpu_sc as plsc

N = 10000
E = 320000
D = 128

NC = 2
NS = 16
NW = NC * NS
NP = 10240
RPT = NP // NS
K = 80
NCH = 128
EPT = NCH * K
EP = NW * EPT
DF = 4
HCH = NCH // 2

_mesh = plsc.VectorSubcoreMesh(core_axis_name="core", subcore_axis_name="subcore")


def _fill(ref, rows, value):
    cols = ref.shape[-1]
    vec = jnp.full((16,), value, dtype=ref.dtype)

    @pl.loop(0, rows)
    def _(i):
        @pl.loop(0, cols, step=16)
        def _(j):
            ref[i, pl.ds(j, 16)] = vec


def _fill_rows(ref, rows, vec16):
    zvec = jnp.zeros((16,), dtype=ref.dtype)

    @pl.loop(0, rows)
    def _(i):
        for j in range(8):
            ref[i, pl.ds(j * 16, 16)] = vec16 if j == 0 else zvec


@functools.partial(
    pl.kernel,
    out_type=jax.ShapeDtypeStruct((NC, NP, D), jnp.float32),
    mesh=_mesh,
    scratch_types=[
        pltpu.VMEM_SHARED((NP, D), jnp.float32),
        pltpu.VMEM((K, D), jnp.float32),
        pltpu.VMEM((K, D), jnp.float32),
        pltpu.VMEM((HCH, K), jnp.int32),
        pltpu.VMEM((HCH, K), jnp.int32),
        pltpu.SemaphoreType.DMA,
    ],
)
def _deg_kernel(src3_hbm, dst3_hbm, cnt_hbm, cnt_sh, ones_s, ones_d,
                sidx, didx, dsem):
    c = lax.axis_index("core")
    s = lax.axis_index("subcore")
    wid = c * NS + s
    _fill(ones_s, K, 0.0)

    @pl.loop(0, RPT // K)
    def _(z):
        pltpu.sync_copy(ones_s, cnt_sh.at[pl.ds(s * RPT + z * K, K)])

    lane = lax.broadcasted_iota(jnp.int32, (16,), 0)
    _fill_rows(ones_s, K, jnp.where(lane == 0, 1.0, 0.0).astype(jnp.float32))
    _fill_rows(ones_d, K, jnp.where(lane == 1, 1.0, 0.0).astype(jnp.float32))
    plsc.subcore_barrier()

    for half in range(2):
        base = half * HCH
        pltpu.sync_copy(src3_hbm.at[wid, pl.ds(base, HCH)], sidx)
        pltpu.sync_copy(dst3_hbm.at[wid, pl.ds(base, HCH)], didx)

        @pl.loop(0, HCH // DF)
        def _(u):
            descs = []
            for q in range(DF):
                ci = u * DF + q
                descs.append(pltpu.async_copy(
                    ones_s, cnt_sh.at[sidx.at[ci]], dsem, add=True))
                descs.append(pltpu.async_copy(
                    ones_d, cnt_sh.at[didx.at[ci]], dsem, add=True))
            for dd in descs:
                dd.wait()

    plsc.subcore_barrier()
    rows = pl.ds(s * RPT, RPT)
    pltpu.sync_copy(cnt_sh.at[rows], cnt_hbm.at[c, rows])


@functools.partial(
    pl.kernel,
    out_type=jax.ShapeDtypeStruct((NC, NP, D), jnp.float32),
    mesh=_mesh,
    scratch_types=[
        pltpu.VMEM_SHARED((NP, D), jnp.float32),
        pltpu.VMEM((2, K, D), jnp.float32),
        pltpu.VMEM((EPT,), jnp.int32),
        pltpu.VMEM((NCH, K), jnp.int32),
        pltpu.SemaphoreType.DMA,
        pltpu.SemaphoreType.DMA,
        pltpu.SemaphoreType.DMA,
    ],
)
def _agg_kernel(h_hbm, src1_hbm, dst3_hbm, pagg_hbm, agg_sh, rows_v,
                sidx, didx, isem, gsem, ssem):
    c = lax.axis_index("core")
    s = lax.axis_index("subcore")
    wid = c * NS + s
    lds = pltpu.async_copy(src1_hbm.at[pl.ds(wid * EPT, EPT)], sidx, isem)
    ldd = pltpu.async_copy(dst3_hbm.at[wid], didx, isem)

    _fill(rows_v.at[0], K, 0.0)

    @pl.loop(0, RPT // K)
    def _(z):
        pltpu.sync_copy(rows_v.at[0], agg_sh.at[pl.ds(s * RPT + z * K, K)])

    lds.wait()
    ldd.wait()
    plsc.subcore_barrier()

    def g_start(ci, b):
        return pltpu.async_copy(h_hbm.at[sidx.at[pl.ds(ci * K, K)]],
                                rows_v.at[b], gsem)

    def s_start(ci, b):
        return pltpu.async_copy(rows_v.at[b], agg_sh.at[didx.at[ci]], ssem,
                                add=True)

    g_start(0, 0).wait()

    @pl.loop(0, NCH - 1)
    def _(c):
        b = c % 2
        gd = g_start(c + 1, 1 - b)
        sd = s_start(c, b)
        sd.wait()
        gd.wait()

    s_start(NCH - 1, (NCH - 1) % 2).wait()

    plsc.subcore_barrier()
    rows = pl.ds(s * RPT, RPT)
    pltpu.sync_copy(agg_sh.at[rows], pagg_hbm.at[c, rows])


def _norm_body(deg_ref, ns_ref, nd_ref):
    cnt = deg_ref[0] + deg_ref[1]
    ns_ref[...] = lax.rsqrt(jnp.maximum(cnt[:, 0:1], 1.0))
    nd_ref[...] = lax.rsqrt(jnp.maximum(cnt[:, 1:2], 1.0))


def _norms(deg):
    return pl.pallas_call(
        _norm_body,
        out_shape=(jax.ShapeDtypeStruct((NP, 1), jnp.float32),
                   jax.ShapeDtypeStruct((NP, 1), jnp.float32)),
    )(deg)


_BR = 1024


def _mm1_body(x_ref, w_ref, ns_ref, o_ref):
    o_ref[...] = jnp.dot(x_ref[...], w_ref[...],
                         preferred_element_type=jnp.float32) * ns_ref[...]


def _mm1(x, W1, ns):
    return pl.pallas_call(
        _mm1_body,
        grid=(NP // _BR,),
        in_specs=[
            pl.BlockSpec((_BR, D), lambda i: (i, 0)),
            pl.BlockSpec((D, D), lambda i: (0, 0)),
            pl.BlockSpec((_BR, 1), lambda i: (i, 0)),
        ],
        out_specs=pl.BlockSpec((_BR, D), lambda i: (i, 0)),
        out_shape=jax.ShapeDtypeStruct((NP, D), jnp.float32),
    )(x, W1, ns)


def _mid_body(p_ref, nd_ref, b_ref, w_ref, ns_ref, o_ref):
    h = (p_ref[0] + p_ref[1]) * nd_ref[...] + b_ref[...]
    h = jnp.maximum(h, 0.0)
    o_ref[...] = jnp.dot(h, w_ref[...],
                         preferred_element_type=jnp.float32) * ns_ref[...]


def _mid(pagg, nd, b1, W2, ns):
    return pl.pallas_call(
        _mid_body,
        grid=(NP // _BR,),
        in_specs=[
            pl.BlockSpec((NC, _BR, D), lambda i: (0, i, 0)),
            pl.BlockSpec((_BR, 1), lambda i: (i, 0)),
            pl.BlockSpec((1, D), lambda i: (0, 0)),
            pl.BlockSpec((D, D), lambda i: (0, 0)),
            pl.BlockSpec((_BR, 1), lambda i: (i, 0)),
        ],
        out_specs=pl.BlockSpec((_BR, D), lambda i: (i, 0)),
        out_shape=jax.ShapeDtypeStruct((NP, D), jnp.float32),
    )(pagg, nd, b1, W2, ns)


def _fin_body(p_ref, nd_ref, b_ref, wt_ref, bfc_ref, o_ref):
    h = (p_ref[0] + p_ref[1]) * nd_ref[...] + b_ref[...]
    o_ref[...] = jnp.dot(h, wt_ref[...],
                         preferred_element_type=jnp.float32) + bfc_ref[...]


def _fin(pagg, nd, b2, WfcT, bfc):
    return pl.pallas_call(
        _fin_body,
        grid=(NP // _BR,),
        in_specs=[
            pl.BlockSpec((NC, _BR, D), lambda i: (0, i, 0)),
            pl.BlockSpec((_BR, 1), lambda i: (i, 0)),
            pl.BlockSpec((1, D), lambda i: (0, 0)),
            pl.BlockSpec((D, D), lambda i: (0, 0)),
            pl.BlockSpec((1, D), lambda i: (0, 0)),
        ],
        out_specs=pl.BlockSpec((_BR, D), lambda i: (i, 0)),
        out_shape=jax.ShapeDtypeStruct((N, D), jnp.float32),
    )(pagg, nd, b2, WfcT, bfc)


def kernel(in_feat, edge_index, W1, b1, W2, b2, Wfc, bfc):
    pad1 = N + jax.lax.rem(jnp.arange(EP - E, dtype=edge_index.dtype), NP - N)
    pad = jnp.broadcast_to(pad1, (2, EP - E))
    ep = jnp.concatenate([edge_index, pad], axis=1)
    src1 = ep[0]
    src3 = ep[0].reshape(NW, NCH, K)
    dst3 = ep[1].reshape(NW, NCH, K)
    deg = _deg_kernel(src3, dst3)
    ns, nd = _norms(deg)
    h1 = _mm1(in_feat, W1, ns)
    p1 = _agg_kernel(h1, src1, dst3)
    h2 = _mid(p1, nd, b1.reshape(1, D), W2, ns)
    p2 = _agg_kernel(h2, src1, dst3)
    out = _fin(p2, nd, b2.reshape(1, D), Wfc.T, bfc.reshape(1, D))
    return out


# speedup vs baseline: 2.4366x; 1.1626x over previous
import functools

import jax
import jax.numpy as jnp
from jax import lax
from jax.experimental import pallas as pl
from jax.experimental.pallas import tpu as pltpu
from jax.experimental.pallas import tpu_sc as plsc

N = 10000
E = 320000
D = 128

NC = 2
NS = 16
NW = NC * NS
NP = 10240
RPT = NP // NS
K = 128
NCH = 80
EPT = NCH * K
EP = NW * EPT
DF = 4
HCH = NCH // 2

_mesh = plsc.VectorSubcoreMesh(core_axis_name="core", subcore_axis_name="subcore")


def _fill(ref, rows, value):
    cols = ref.shape[-1]
    vec = jnp.full((16,), value, dtype=ref.dtype)

    @pl.loop(0, rows)
    def _(i):
        @pl.loop(0, cols, step=16)
        def _(j):
            ref[i, pl.ds(j, 16)] = vec


def _fill_rows(ref, rows, vec16):
    zvec = jnp.zeros((16,), dtype=ref.dtype)

    @pl.loop(0, rows)
    def _(i):
        for j in range(8):
            ref[i, pl.ds(j * 16, 16)] = vec16 if j == 0 else zvec


@functools.partial(
    pl.kernel,
    out_type=jax.ShapeDtypeStruct((NC, NP, D), jnp.float32),
    mesh=_mesh,
    scratch_types=[
        pltpu.VMEM_SHARED((NP, D), jnp.float32),
        pltpu.VMEM((K, D), jnp.float32),
        pltpu.VMEM((K, D), jnp.float32),
        pltpu.VMEM((HCH, K), jnp.int32),
        pltpu.VMEM((HCH, K), jnp.int32),
        pltpu.SemaphoreType.DMA,
    ],
)
def _deg_kernel(src3_hbm, dst3_hbm, cnt_hbm, cnt_sh, ones_s, ones_d,
                sidx, didx, dsem):
    c = lax.axis_index("core")
    s = lax.axis_index("subcore")
    wid = c * NS + s
    _fill(ones_s, K, 0.0)

    @pl.loop(0, RPT // K)
    def _(z):
        pltpu.sync_copy(ones_s, cnt_sh.at[pl.ds(s * RPT + z * K, K)])

    lane = lax.broadcasted_iota(jnp.int32, (16,), 0)
    _fill_rows(ones_s, K, jnp.where(lane == 0, 1.0, 0.0).astype(jnp.float32))
    _fill_rows(ones_d, K, jnp.where(lane == 1, 1.0, 0.0).astype(jnp.float32))
    plsc.subcore_barrier()

    for half in range(2):
        base = half * HCH
        pltpu.sync_copy(src3_hbm.at[wid, pl.ds(base, HCH)], sidx)
        pltpu.sync_copy(dst3_hbm.at[wid, pl.ds(base, HCH)], didx)

        @pl.loop(0, HCH // DF)
        def _(u):
            descs = []
            for q in range(DF):
                ci = u * DF + q
                descs.append(pltpu.async_copy(
                    ones_s, cnt_sh.at[sidx.at[ci]], dsem, add=True))
                descs.append(pltpu.async_copy(
                    ones_d, cnt_sh.at[didx.at[ci]], dsem, add=True))
            for dd in descs:
                dd.wait()

    plsc.subcore_barrier()
    rows = pl.ds(s * RPT, RPT)
    pltpu.sync_copy(cnt_sh.at[rows], cnt_hbm.at[c, rows])


@functools.partial(
    pl.kernel,
    out_type=jax.ShapeDtypeStruct((NC, NP, D), jnp.float32),
    mesh=_mesh,
    scratch_types=[
        pltpu.VMEM_SHARED((NP, D), jnp.float32),
        pltpu.VMEM((2, K, D), jnp.float32),
        pltpu.VMEM((EPT,), jnp.int32),
        pltpu.VMEM((HCH, K), jnp.int32),
        pltpu.SemaphoreType.DMA,
        pltpu.SemaphoreType.DMA,
        pltpu.SemaphoreType.DMA,
    ],
)
def _agg_kernel(h_hbm, src1_hbm, dst3_hbm, pagg_hbm, agg_sh, rows_v,
                sidx, didx, isem, gsem, ssem):
    c = lax.axis_index("core")
    s = lax.axis_index("subcore")
    wid = c * NS + s
    lds = pltpu.async_copy(src1_hbm.at[pl.ds(wid * EPT, EPT)], sidx, isem)
    ldd = pltpu.async_copy(dst3_hbm.at[wid, pl.ds(0, HCH)], didx, isem)

    _fill(rows_v.at[0], K, 0.0)

    @pl.loop(0, RPT // K)
    def _(z):
        pltpu.sync_copy(rows_v.at[0], agg_sh.at[pl.ds(s * RPT + z * K, K)])

    lds.wait()
    ldd.wait()
    plsc.subcore_barrier()

    def g_start(ci, b):
        return pltpu.async_copy(h_hbm.at[sidx.at[pl.ds(ci * K, K)]],
                                rows_v.at[b], gsem)

    def s_start(ci, b, lci):
        return pltpu.async_copy(rows_v.at[b], agg_sh.at[didx.at[lci]], ssem,
                                add=True)

    g_start(0, 0).wait()

    @pl.loop(0, HCH)
    def _(c):
        b = c % 2
        gd = g_start(c + 1, 1 - b)
        sd = s_start(c, b, c)
        sd.wait()
        gd.wait()

    pltpu.sync_copy(dst3_hbm.at[wid, pl.ds(HCH, HCH)], didx)

    @pl.loop(HCH, NCH - 1)
    def _(c):
        b = c % 2
        gd = g_start(c + 1, 1 - b)
        sd = s_start(c, b, c - HCH)
        sd.wait()
        gd.wait()

    s_start(NCH - 1, (NCH - 1) % 2, NCH - 1 - HCH).wait()

    plsc.subcore_barrier()
    rows = pl.ds(s * RPT, RPT)
    pltpu.sync_copy(agg_sh.at[rows], pagg_hbm.at[c, rows])


def _norm_body(deg_ref, ns_ref, nd_ref):
    cnt = deg_ref[0] + deg_ref[1]
    ns_ref[...] = lax.rsqrt(jnp.maximum(cnt[:, 0:1], 1.0))
    nd_ref[...] = lax.rsqrt(jnp.maximum(cnt[:, 1:2], 1.0))


def _norms(deg):
    return pl.pallas_call(
        _norm_body,
        out_shape=(jax.ShapeDtypeStruct((NP, 1), jnp.float32),
                   jax.ShapeDtypeStruct((NP, 1), jnp.float32)),
    )(deg)


_BR = 1024


def _mm1_body(x_ref, w_ref, ns_ref, o_ref):
    o_ref[...] = jnp.dot(x_ref[...], w_ref[...],
                         preferred_element_type=jnp.float32) * ns_ref[...]


def _mm1(x, W1, ns):
    return pl.pallas_call(
        _mm1_body,
        grid=(NP // _BR,),
        in_specs=[
            pl.BlockSpec((_BR, D), lambda i: (i, 0)),
            pl.BlockSpec((D, D), lambda i: (0, 0)),
            pl.BlockSpec((_BR, 1), lambda i: (i, 0)),
        ],
        out_specs=pl.BlockSpec((_BR, D), lambda i: (i, 0)),
        out_shape=jax.ShapeDtypeStruct((NP, D), jnp.float32),
    )(x, W1, ns)


def _mid_body(p_ref, nd_ref, b_ref, w_ref, ns_ref, o_ref):
    h = (p_ref[0] + p_ref[1]) * nd_ref[...] + b_ref[...]
    h = jnp.maximum(h, 0.0)
    o_ref[...] = jnp.dot(h, w_ref[...],
                         preferred_element_type=jnp.float32) * ns_ref[...]


def _mid(pagg, nd, b1, W2, ns):
    return pl.pallas_call(
        _mid_body,
        grid=(NP // _BR,),
        in_specs=[
            pl.BlockSpec((NC, _BR, D), lambda i: (0, i, 0)),
            pl.BlockSpec((_BR, 1), lambda i: (i, 0)),
            pl.BlockSpec((1, D), lambda i: (0, 0)),
            pl.BlockSpec((D, D), lambda i: (0, 0)),
            pl.BlockSpec((_BR, 1), lambda i: (i, 0)),
        ],
        out_specs=pl.BlockSpec((_BR, D), lambda i: (i, 0)),
        out_shape=jax.ShapeDtypeStruct((NP, D), jnp.float32),
    )(pagg, nd, b1, W2, ns)


def _fin_body(p_ref, nd_ref, b_ref, wt_ref, bfc_ref, o_ref):
    h = (p_ref[0] + p_ref[1]) * nd_ref[...] + b_ref[...]
    o_ref[...] = jnp.dot(h, wt_ref[...],
                         preferred_element_type=jnp.float32) + bfc_ref[...]


def _fin(pagg, nd, b2, WfcT, bfc):
    return pl.pallas_call(
        _fin_body,
        grid=(NP // _BR,),
        in_specs=[
            pl.BlockSpec((NC, _BR, D), lambda i: (0, i, 0)),
            pl.BlockSpec((_BR, 1), lambda i: (i, 0)),
            pl.BlockSpec((1, D), lambda i: (0, 0)),
            pl.BlockSpec((D, D), lambda i: (0, 0)),
            pl.BlockSpec((1, D), lambda i: (0, 0)),
        ],
        out_specs=pl.BlockSpec((_BR, D), lambda i: (i, 0)),
        out_shape=jax.ShapeDtypeStruct((N, D), jnp.float32),
    )(pagg, nd, b2, WfcT, bfc)


def kernel(in_feat, edge_index, W1, b1, W2, b2, Wfc, bfc):
    pad1 = N + jax.lax.rem(jnp.arange(EP - E, dtype=edge_index.dtype), NP - N)
    pad = jnp.broadcast_to(pad1, (2, EP - E))
    ep = jnp.concatenate([edge_index, pad], axis=1)
    src1 = ep[0]
    src3 = ep[0].reshape(NW, NCH, K)
    dst3 = ep[1].reshape(NW, NCH, K)
    deg = _deg_kernel(src3, dst3)
    ns, nd = _norms(deg)
    h1 = _mm1(in_feat, W1, ns)
    p1 = _agg_kernel(h1, src1, dst3)
    h2 = _mid(p1, nd, b1.reshape(1, D), W2, ns)
    p2 = _agg_kernel(h2, src1, dst3)
    out = _fin(p2, nd, b2.reshape(1, D), Wfc.T, bfc.reshape(1, D))
    return out


# speedup vs baseline: 2.4744x; 1.0155x over previous
import functools

import jax
import jax.numpy as jnp
from jax import lax
from jax.experimental import pallas as pl
from jax.experimental.pallas import tpu as pltpu
from jax.experimental.pallas import tpu_sc as plsc

N = 10000
E = 320000
D = 128

NC = 2
NS = 16
NW = NC * NS
NP = 10240
RPT = NP // NS
K = 128
NCH = 80
EPT = NCH * K
EP = NW * EPT
DF = 4
HCH = NCH // 2

_mesh = plsc.VectorSubcoreMesh(core_axis_name="core", subcore_axis_name="subcore")


def _fill(ref, rows, value):
    cols = ref.shape[-1]
    vec = jnp.full((16,), value, dtype=ref.dtype)

    @pl.loop(0, rows)
    def _(i):
        @pl.loop(0, cols, step=16)
        def _(j):
            ref[i, pl.ds(j, 16)] = vec


def _fill_rows(ref, rows, vec16):
    zvec = jnp.zeros((16,), dtype=ref.dtype)

    @pl.loop(0, rows)
    def _(i):
        for j in range(8):
            ref[i, pl.ds(j * 16, 16)] = vec16 if j == 0 else zvec


@functools.partial(
    pl.kernel,
    out_type=jax.ShapeDtypeStruct((NC, NP, D), jnp.float32),
    mesh=_mesh,
    scratch_types=[
        pltpu.VMEM_SHARED((NP, D), jnp.float32),
        pltpu.VMEM((K, D), jnp.float32),
        pltpu.VMEM((K, D), jnp.float32),
        pltpu.VMEM((HCH, K), jnp.int32),
        pltpu.VMEM((HCH, K), jnp.int32),
        pltpu.SemaphoreType.DMA,
    ],
)
def _deg_kernel(src3_hbm, dst3_hbm, cnt_hbm, cnt_sh, ones_s, ones_d,
                sidx, didx, dsem):
    c = lax.axis_index("core")
    s = lax.axis_index("subcore")
    wid = c * NS + s
    _fill(ones_s, K, 0.0)

    @pl.loop(0, RPT // K)
    def _(z):
        pltpu.sync_copy(ones_s, cnt_sh.at[pl.ds(s * RPT + z * K, K)])

    lane = lax.broadcasted_iota(jnp.int32, (16,), 0)
    _fill_rows(ones_s, K, jnp.where(lane == 0, 1.0, 0.0).astype(jnp.float32))
    _fill_rows(ones_d, K, jnp.where(lane == 1, 1.0, 0.0).astype(jnp.float32))
    plsc.subcore_barrier()

    for half in range(2):
        base = half * HCH
        pltpu.sync_copy(src3_hbm.at[wid, pl.ds(base, HCH)], sidx)
        pltpu.sync_copy(dst3_hbm.at[wid, pl.ds(base, HCH)], didx)

        @pl.loop(0, HCH // DF)
        def _(u):
            descs = []
            for q in range(DF):
                ci = u * DF + q
                descs.append(pltpu.async_copy(
                    ones_s, cnt_sh.at[sidx.at[ci]], dsem, add=True))
                descs.append(pltpu.async_copy(
                    ones_d, cnt_sh.at[didx.at[ci]], dsem, add=True))
            for dd in descs:
                dd.wait()

    plsc.subcore_barrier()
    rows = pl.ds(s * RPT, RPT)
    pltpu.sync_copy(cnt_sh.at[rows], cnt_hbm.at[c, rows])


@functools.partial(
    pl.kernel,
    out_type=jax.ShapeDtypeStruct((NC, NP, D), jnp.float32),
    mesh=_mesh,
    scratch_types=[
        pltpu.VMEM_SHARED((NP, D), jnp.float32),
        pltpu.VMEM((2, K, D), jnp.float32),
        pltpu.VMEM((EPT,), jnp.int32),
        pltpu.VMEM((HCH, K), jnp.int32),
        pltpu.SemaphoreType.DMA,
        pltpu.SemaphoreType.DMA,
        pltpu.SemaphoreType.DMA,
    ],
)
def _agg_kernel(h_hbm, src1_hbm, dst3_hbm, pagg_hbm, agg_sh, rows_v,
                sidx, didx, isem, gsem, ssem):
    c = lax.axis_index("core")
    s = lax.axis_index("subcore")
    wid = c * NS + s
    lds = pltpu.async_copy(src1_hbm.at[pl.ds(wid * EPT, EPT)], sidx, isem)
    ldd = pltpu.async_copy(dst3_hbm.at[wid, pl.ds(0, HCH)], didx, isem)

    _fill(rows_v.at[0], K, 0.0)

    @pl.loop(0, RPT // K)
    def _(z):
        pltpu.sync_copy(rows_v.at[0], agg_sh.at[pl.ds(s * RPT + z * K, K)])

    lds.wait()
    ldd.wait()
    plsc.subcore_barrier()

    def g_start(ci, b):
        return pltpu.async_copy(h_hbm.at[sidx.at[pl.ds(ci * K, K)]],
                                rows_v.at[b], gsem)

    def s_start(ci, b, lci):
        return pltpu.async_copy(rows_v.at[b], agg_sh.at[didx.at[lci]], ssem,
                                add=True)

    g_start(0, 0).wait()

    @pl.loop(0, HCH)
    def _(c):
        b = c % 2
        gd = g_start(c + 1, 1 - b)
        sd = s_start(c, b, c)
        sd.wait()
        gd.wait()

    pltpu.sync_copy(dst3_hbm.at[wid, pl.ds(HCH, HCH)], didx)

    @pl.loop(HCH, NCH - 1)
    def _(c):
        b = c % 2
        gd = g_start(c + 1, 1 - b)
        sd = s_start(c, b, c - HCH)
        sd.wait()
        gd.wait()

    s_start(NCH - 1, (NCH - 1) % 2, NCH - 1 - HCH).wait()

    plsc.subcore_barrier()
    rows = pl.ds(s * RPT, RPT)
    pltpu.sync_copy(agg_sh.at[rows], pagg_hbm.at[c, rows])


def _norm_body(deg_ref, ns_ref, nd_ref):
    cnt = deg_ref[0] + deg_ref[1]
    ns_ref[...] = lax.rsqrt(jnp.maximum(cnt[:, 0:1], 1.0))
    nd_ref[...] = lax.rsqrt(jnp.maximum(cnt[:, 1:2], 1.0))


def _norms(deg):
    return pl.pallas_call(
        _norm_body,
        out_shape=(jax.ShapeDtypeStruct((NP, 1), jnp.float32),
                   jax.ShapeDtypeStruct((NP, 1), jnp.float32)),
    )(deg)


_BR = 1024


def _mm1_body(deg_ref, x_ref, w_ref, o_ref, ns_ref, nd_ref):
    cnt = deg_ref[0] + deg_ref[1]
    ns = lax.rsqrt(jnp.maximum(cnt[:, 0:1], 1.0))
    ns_ref[...] = ns
    nd_ref[...] = lax.rsqrt(jnp.maximum(cnt[:, 1:2], 1.0))
    o_ref[...] = jnp.dot(x_ref[...], w_ref[...],
                         preferred_element_type=jnp.float32) * ns


def _mm1(deg, x, W1):
    return pl.pallas_call(
        _mm1_body,
        grid=(NP // _BR,),
        in_specs=[
            pl.BlockSpec((NC, _BR, D), lambda i: (0, i, 0)),
            pl.BlockSpec((_BR, D), lambda i: (i, 0)),
            pl.BlockSpec((D, D), lambda i: (0, 0)),
        ],
        out_specs=(pl.BlockSpec((_BR, D), lambda i: (i, 0)),
                   pl.BlockSpec((_BR, 1), lambda i: (i, 0)),
                   pl.BlockSpec((_BR, 1), lambda i: (i, 0))),
        out_shape=(jax.ShapeDtypeStruct((NP, D), jnp.float32),
                   jax.ShapeDtypeStruct((NP, 1), jnp.float32),
                   jax.ShapeDtypeStruct((NP, 1), jnp.float32)),
    )(deg, x, W1)


def _mid_body(p_ref, nd_ref, b_ref, w_ref, ns_ref, o_ref):
    h = (p_ref[0] + p_ref[1]) * nd_ref[...] + b_ref[...]
    h = jnp.maximum(h, 0.0)
    o_ref[...] = jnp.dot(h, w_ref[...],
                         preferred_element_type=jnp.float32) * ns_ref[...]


def _mid(pagg, nd, b1, W2, ns):
    return pl.pallas_call(
        _mid_body,
        grid=(NP // _BR,),
        in_specs=[
            pl.BlockSpec((NC, _BR, D), lambda i: (0, i, 0)),
            pl.BlockSpec((_BR, 1), lambda i: (i, 0)),
            pl.BlockSpec((1, D), lambda i: (0, 0)),
            pl.BlockSpec((D, D), lambda i: (0, 0)),
            pl.BlockSpec((_BR, 1), lambda i: (i, 0)),
        ],
        out_specs=pl.BlockSpec((_BR, D), lambda i: (i, 0)),
        out_shape=jax.ShapeDtypeStruct((NP, D), jnp.float32),
    )(pagg, nd, b1, W2, ns)


def _fin_body(p_ref, nd_ref, b_ref, wt_ref, bfc_ref, o_ref):
    h = (p_ref[0] + p_ref[1]) * nd_ref[...] + b_ref[...]
    o_ref[...] = jnp.dot(h, wt_ref[...],
                         preferred_element_type=jnp.float32) + bfc_ref[...]


def _fin(pagg, nd, b2, WfcT, bfc):
    return pl.pallas_call(
        _fin_body,
        grid=(NP // _BR,),
        in_specs=[
            pl.BlockSpec((NC, _BR, D), lambda i: (0, i, 0)),
            pl.BlockSpec((_BR, 1), lambda i: (i, 0)),
            pl.BlockSpec((1, D), lambda i: (0, 0)),
            pl.BlockSpec((D, D), lambda i: (0, 0)),
            pl.BlockSpec((1, D), lambda i: (0, 0)),
        ],
        out_specs=pl.BlockSpec((_BR, D), lambda i: (i, 0)),
        out_shape=jax.ShapeDtypeStruct((N, D), jnp.float32),
    )(pagg, nd, b2, WfcT, bfc)


def kernel(in_feat, edge_index, W1, b1, W2, b2, Wfc, bfc):
    pad1 = N + jax.lax.rem(jnp.arange(EP - E, dtype=edge_index.dtype), NP - N)
    pad = jnp.broadcast_to(pad1, (2, EP - E))
    ep = jnp.concatenate([edge_index, pad], axis=1)
    src1 = ep[0]
    src3 = ep[0].reshape(NW, NCH, K)
    dst3 = ep[1].reshape(NW, NCH, K)
    deg = _deg_kernel(src3, dst3)
    h1, ns, nd = _mm1(deg, in_feat, W1)
    p1 = _agg_kernel(h1, src1, dst3)
    h2 = _mid(p1, nd, b1.reshape(1, D), W2, ns)
    p2 = _agg_kernel(h2, src1, dst3)
    out = _fin(p2, nd, b2.reshape(1, D), Wfc.T, bfc.reshape(1, D))
    return out
